# baseline (speedup 1.0000x reference)
import functools

import jax
import jax.numpy as jnp
from jax import lax
from jax.experimental import pallas as pl
from jax.experimental.pallas import tpu as pltpu
from jax.experimental.pallas import tpu_sc as plsc

_H = 128
_B = 16
_HW = 64
_NT = 16
_CH = 448
_STR = 80
_RU = 8
_RG = 1000


def _sc_pool_means(cpg2, cb, mirna2, mb):
    n_c = cb.shape[0]
    n_m = mb.shape[0]

    mesh = plsc.VectorSubcoreMesh(core_axis_name="c", subcore_axis_name="s")

    @functools.partial(
        pl.kernel,
        mesh=mesh,
        compiler_params=pltpu.CompilerParams(needs_layout_passes=False,
                                             use_tc_tiling_on_sc=False),
        out_type=[jax.ShapeDtypeStruct((2 * _B * _H,), jnp.float32)] * 2,
        scratch_types=[
            pltpu.VMEM((_CH, _HW), jnp.float32),
            pltpu.VMEM((_CH, _HW), jnp.float32),
            pltpu.VMEM((_CH + 16,), jnp.int32),
            pltpu.VMEM((_CH + 16,), jnp.int32),
            pltpu.VMEM((2 * _B, _STR), jnp.float32),
            pltpu.VMEM((_B,), jnp.int32),
            pltpu.VMEM((_STR,), jnp.float32),
            pltpu.VMEM((_HW,), jnp.float32),
            pltpu.VMEM_SHARED((_B, _STR), jnp.float32),
            pltpu.VMEM_SHARED((_B, _STR), jnp.float32),
            pltpu.SemaphoreType.DMA,
            pltpu.SemaphoreType.DMA,
        ],
    )
    def sc_run(cpg_h, cb_h, mirna_h, mb_h, dna_h, mir_h,
               dbufA, dbufB, bbufA, bbufB, acc, iref, rowbuf, sbuf,
               szc, szm, semA, semB):
        sid = lax.axis_index("s")
        core = lax.axis_index("c")
        col0 = core * _HW
        off = lax.broadcasted_iota(jnp.int32, (16,), 0)
        ones = jnp.ones((16,), jnp.float32)
        zeros = jnp.zeros((16,), jnp.float32)
        mask0 = off == 0
        bufsA = (dbufA, bbufA, semA)
        bufsB = (dbufB, bbufB, semB)

        iref[...] = off

        def zrow(t, carry):
            row = t // (_STR // 16)
            grp = t % (_STR // 16)
            plsc.store_scatter(acc, [jnp.full((16,), row, jnp.int32),
                                     grp * 16 + off], zeros)
            return carry
        lax.fori_loop(0, 2 * _B * (_STR // 16), zrow, 0)
        for t in range(_STR // 16):
            rowbuf[pl.ds(t * 16, 16)] = zeros
        pltpu.sync_copy(rowbuf, szc.at[sid])
        pltpu.sync_copy(rowbuf, szm.at[sid])
        plsc.subcore_barrier()

        def flush_regs(av, seg_row, cnt):
            for j in range(4):
                plsc.addupdate_scatter(acc, [seg_row, (j * 16) + off], av[j])
            plsc.addupdate_scatter(acc, [seg_row, jnp.full((16,), _HW,
                                                           jnp.int32)],
                                   cnt, mask=mask0)

        def process(data_h, batch_h, n_rows, seg_base):
            nchunks = -(-n_rows // _CH)
            t_steps = -(-nchunks // _NT)

            def start_into(refs, c):
                dbuf, bbuf, sem = refs
                s2 = jnp.minimum(c * _CH, n_rows - _CH)
                pltpu.async_copy(data_h.at[pl.ds(s2, _CH), pl.ds(col0, _HW)],
                                 dbuf, sem)
                pltpu.async_copy(batch_h.at[pl.ds(s2, _CH)],
                                 bbuf.at[pl.ds(0, _CH)], sem)

            def wait_into(refs):
                dbuf, bbuf, sem = refs
                pltpu.make_async_copy(
                    data_h.at[pl.ds(0, _CH), pl.ds(0, _HW)], dbuf, sem).wait()
                pltpu.make_async_copy(batch_h.at[pl.ds(0, _CH)],
                                      bbuf.at[pl.ds(0, _CH)], sem).wait()

            def row_scatter(dbuf, bbuf, r_lo, r_hi):
                def row(i, carry):
                    bs = plsc.load_gather(bbuf,
                                          [jnp.full((16,), i, jnp.int32)])
                    seg_row = bs + seg_base
                    for j in range(4):
                        x = dbuf[i, pl.ds(j * 16, 16)]
                        plsc.addupdate_scatter(acc, [seg_row, (j * 16) + off],
                                               x)
                    plsc.addupdate_scatter(
                        acc, [seg_row, jnp.full((16,), _HW, jnp.int32)],
                        ones, mask=mask0)
                    return carry
                lax.fori_loop(r_lo, r_hi, row, 0)

            def chunk_work(dbuf, bbuf, c):
                s2 = jnp.minimum(c * _CH, n_rows - _CH)
                lo = c * _CH - s2
                v = bbuf[pl.ds(0, 16)]
                bmin = v
                bmax = v
                for t in range(1, _CH // 16):
                    v = bbuf[pl.ds(t * 16, 16)]
                    bmin = jnp.minimum(bmin, v)
                    bmax = jnp.maximum(bmax, v)
                bmin_s = jnp.min(bmin)
                bmax_s = jnp.max(bmax)
                uniform = jnp.logical_and(lo == 0, bmin_s == bmax_s)

                def fast():
                    def body(i, carry):
                        av = list(carry)
                        r0 = i * _RU
                        for u in range(_RU):
                            for j in range(4):
                                av[j] = av[j] + dbuf[r0 + u,
                                                     pl.ds(j * 16, 16)]
                        return tuple(av)
                    av = lax.fori_loop(0, _CH // _RU, body,
                                       tuple([zeros] * 4))
                    flush_regs(av, jnp.full((16,), bmin_s + seg_base,
                                            jnp.int32),
                               jnp.full((16,), float(_CH), jnp.float32))

                def slow():
                    def sub(g, carry):
                        r0 = g * _RU
                        bvec = bbuf[pl.ds(r0, 16)]
                        big = jnp.full((16,), 2 ** 30, jnp.int32)
                        small = jnp.full((16,), -2 ** 30, jnp.int32)
                        sel = off < _RU
                        mn = jnp.min(jnp.where(sel, bvec, big))
                        mx = jnp.max(jnp.where(sel, bvec, small))
                        ok = jnp.logical_and(mn == mx, r0 >= lo)

                        def gfast():
                            av = [zeros] * 4
                            for u in range(_RU):
                                for j in range(4):
                                    av[j] = av[j] + dbuf[r0 + u,
                                                         pl.ds(j * 16, 16)]
                            flush_regs(av, jnp.full((16,), mn + seg_base,
                                                    jnp.int32),
                                       jnp.full((16,), float(_RU),
                                                jnp.float32))

                        def grows():
                            row_scatter(dbuf, bbuf, jnp.maximum(lo, r0),
                                        r0 + _RU)

                        lax.cond(ok, gfast, grows)
                        return carry
                    lax.fori_loop(lo // _RU, _CH // _RU, sub, 0)

                lax.cond(uniform, fast, slow)

            def body(k, carry):
                c = sid + k * _NT

                def run(cur, nxt):
                    pl.when(k + 1 < t_steps)(lambda: start_into(nxt, c + _NT))
                    wait_into(cur)
                    dbuf, bbuf, _ = cur
                    bbuf[pl.ds(_CH, 16)] = bbuf[pl.ds(_CH - 16, 16)]
                    pl.when(c < nchunks)(lambda: chunk_work(dbuf, bbuf, c))

                lax.cond(k % 2 == 0,
                         lambda: run(bufsA, bufsB),
                         lambda: run(bufsB, bufsA))
                return carry

            start_into(bufsA, sid)
            lax.fori_loop(0, t_steps, body, 0)

        process(cpg_h, cb_h, n_c, 0)
        process(mirna_h, mb_h, n_m, _B)

        pltpu.sync_copy(acc.at[pl.ds(0, _B)], szc.at[iref], add=True)
        pltpu.sync_copy(acc.at[pl.ds(_B, _B)], szm.at[iref], add=True)
        plsc.subcore_barrier()

        def emit(shared, out_h):
            pltpu.sync_copy(shared.at[sid], rowbuf)
            cntv = plsc.load_gather(rowbuf,
                                    [jnp.full((16,), _HW, jnp.int32)])
            den = jnp.maximum(cntv, 1.0)
            for j in range(4):
                sbuf[pl.ds(j * 16, 16)] = rowbuf[pl.ds(j * 16, 16)] / den
            pltpu.sync_copy(sbuf,
                            out_h.at[pl.ds(core * (_B * _H) + sid * _H,
                                           _HW)])

        emit(szc, dna_h)
        emit(szm, mir_h)

    return sc_run(cpg2, cb, mirna2, mb)


def _gene_body(b_ref, x_ref, wmt_ref, wct_ref, bm_ref, bc_ref,
               mrna_ref, cnv_ref, gsum, gcnt):
    i = pl.program_id(0)
    k_steps = pl.num_programs(0)

    @pl.when(i == 0)
    def _init():
        gsum[...] = jnp.zeros_like(gsum)
        gcnt[...] = jnp.zeros_like(gcnt)

    seg_ids = lax.broadcasted_iota(jnp.int32, (_B, _RG), 0)
    oh = (seg_ids == b_ref[0]).astype(jnp.float32)
    gsum[...] += jnp.dot(oh, x_ref[...], preferred_element_type=jnp.float32)
    gcnt[...] += jnp.sum(oh, axis=1, keepdims=True)

    @pl.when(i == k_steps - 1)
    def _fin():
        cnt = gcnt[:, 0:1]
        mean = gsum[...] / jnp.maximum(cnt, 1.0)
        mask = (cnt > 0.0).astype(jnp.float32)
        mrna_ref[...] = (jnp.dot(mean, wmt_ref[...],
                                 preferred_element_type=jnp.float32)
                         + bm_ref[...] * mask)
        cnv_ref[...] = (jnp.dot(mean, wct_ref[...],
                                preferred_element_type=jnp.float32)
                        + bc_ref[...] * mask)


def _gene_project(gene, gb3, wmt, wct, bm2, bc2):
    k_steps = gene.shape[0] // _RG
    return pl.pallas_call(
        _gene_body,
        grid=(k_steps,),
        in_specs=[
            pl.BlockSpec((1, 1, _RG), lambda i: (i, 0, 0)),
            pl.BlockSpec((_RG, _H), lambda i: (i, 0)),
            pl.BlockSpec((_H, _H), lambda i: (0, 0)),
            pl.BlockSpec((_H, _H), lambda i: (0, 0)),
            pl.BlockSpec((1, _H), lambda i: (0, 0)),
            pl.BlockSpec((1, _H), lambda i: (0, 0)),
        ],
        out_specs=[
            pl.BlockSpec((_B, _H), lambda i: (0, 0)),
            pl.BlockSpec((_B, _H), lambda i: (0, 0)),
        ],
        out_shape=[jax.ShapeDtypeStruct((_B, _H), jnp.float32)] * 2,
        scratch_shapes=[
            pltpu.VMEM((_B, _H), jnp.float32),
            pltpu.VMEM((_B, _H), jnp.float32),
        ],
    )(gb3, gene, wmt, wct, bm2, bc2)


def kernel(gene, cpg, mirna, gene_batch, cpg_batch, mirna_batch, Wm, bm, Wc, bc):
    gene = gene.astype(jnp.float32)
    cpg = cpg.astype(jnp.float32)
    mirna = mirna.astype(jnp.float32)
    gb = gene_batch.astype(jnp.int32)
    cb = cpg_batch.astype(jnp.int32)
    mb = mirna_batch.astype(jnp.int32)

    dna_f, mir_f = _sc_pool_means(cpg, cb, mirna, mb)
    dna = (dna_f.reshape(2, _B, _H)[:, :, :_HW]
           .transpose(1, 0, 2).reshape(_B, _H))
    mir = (mir_f.reshape(2, _B, _H)[:, :, :_HW]
           .transpose(1, 0, 2).reshape(_B, _H))
    mrna, cnv = _gene_project(
        gene, gb.reshape(-1, 1, _RG),
        Wm.astype(jnp.float32).T, Wc.astype(jnp.float32).T,
        bm.astype(jnp.float32).reshape(1, _H),
        bc.astype(jnp.float32).reshape(1, _H),
    )
    return (mrna, cnv, dna, mir)


# speedup vs baseline: 1.3162x; 1.3162x over previous
import functools

import jax
import jax.numpy as jnp
from jax import lax
from jax.experimental import pallas as pl
from jax.experimental.pallas import tpu as pltpu
from jax.experimental.pallas import tpu_sc as plsc

_H = 128
_B = 16
_HW = 64
_NT = 16
_CH = 448
_STR = 80
_RU = 8
_RG = 2000


def _sc_pool_means(cpg2, cb, mirna2, mb):
    n_c = cb.shape[0]
    n_m = mb.shape[0]

    mesh = plsc.VectorSubcoreMesh(core_axis_name="c", subcore_axis_name="s")

    @functools.partial(
        pl.kernel,
        mesh=mesh,
        compiler_params=pltpu.CompilerParams(needs_layout_passes=False,
                                             use_tc_tiling_on_sc=False),
        out_type=[jax.ShapeDtypeStruct((_B, _H), jnp.float32)] * 2,
        scratch_types=[
            pltpu.VMEM((_CH, _HW), jnp.float32),
            pltpu.VMEM((_CH, _HW), jnp.float32),
            pltpu.VMEM((_CH + 16,), jnp.int32),
            pltpu.VMEM((_CH + 16,), jnp.int32),
            pltpu.VMEM((2 * _B, _STR), jnp.float32),
            pltpu.VMEM((_B,), jnp.int32),
            pltpu.VMEM((_STR,), jnp.float32),
            pltpu.VMEM((_HW,), jnp.float32),
            pltpu.VMEM_SHARED((_B, _STR), jnp.float32),
            pltpu.VMEM_SHARED((_B, _STR), jnp.float32),
            pltpu.SemaphoreType.DMA,
            pltpu.SemaphoreType.DMA,
        ],
    )
    def sc_run(cpg_h, cb_h, mirna_h, mb_h, dna_h, mir_h,
               dbufA, dbufB, bbufA, bbufB, acc, iref, rowbuf, sbuf,
               szc, szm, semA, semB):
        sid = lax.axis_index("s")
        core = lax.axis_index("c")
        col0 = core * _HW
        off = lax.broadcasted_iota(jnp.int32, (16,), 0)
        ones = jnp.ones((16,), jnp.float32)
        zeros = jnp.zeros((16,), jnp.float32)
        mask0 = off == 0
        bufsA = (dbufA, bbufA, semA)
        bufsB = (dbufB, bbufB, semB)

        iref[...] = off

        def zrow(t, carry):
            row = t // (_STR // 16)
            grp = t % (_STR // 16)
            plsc.store_scatter(acc, [jnp.full((16,), row, jnp.int32),
                                     grp * 16 + off], zeros)
            return carry
        lax.fori_loop(0, 2 * _B * (_STR // 16), zrow, 0)
        for t in range(_STR // 16):
            rowbuf[pl.ds(t * 16, 16)] = zeros
        pltpu.sync_copy(rowbuf, szc.at[sid])
        pltpu.sync_copy(rowbuf, szm.at[sid])
        plsc.subcore_barrier()

        def flush_regs(av, seg_row, cnt):
            for j in range(4):
                plsc.addupdate_scatter(acc, [seg_row, (j * 16) + off], av[j])
            plsc.addupdate_scatter(acc, [seg_row, jnp.full((16,), _HW,
                                                           jnp.int32)],
                                   cnt, mask=mask0)

        def process(data_h, batch_h, n_rows, seg_base):
            nchunks = -(-n_rows // _CH)
            t_steps = -(-nchunks // _NT)

            def start_into(refs, c):
                dbuf, bbuf, sem = refs
                s2 = jnp.minimum(c * _CH, n_rows - _CH)
                pltpu.async_copy(data_h.at[pl.ds(s2, _CH), pl.ds(col0, _HW)],
                                 dbuf, sem)
                pltpu.async_copy(batch_h.at[pl.ds(s2, _CH)],
                                 bbuf.at[pl.ds(0, _CH)], sem)

            def wait_into(refs):
                dbuf, bbuf, sem = refs
                pltpu.make_async_copy(
                    data_h.at[pl.ds(0, _CH), pl.ds(0, _HW)], dbuf, sem).wait()
                pltpu.make_async_copy(batch_h.at[pl.ds(0, _CH)],
                                      bbuf.at[pl.ds(0, _CH)], sem).wait()

            def row_scatter(dbuf, bbuf, r_lo, r_hi):
                def row(i, carry):
                    bs = plsc.load_gather(bbuf,
                                          [jnp.full((16,), i, jnp.int32)])
                    seg_row = bs + seg_base
                    for j in range(4):
                        x = dbuf[i, pl.ds(j * 16, 16)]
                        plsc.addupdate_scatter(acc, [seg_row, (j * 16) + off],
                                               x)
                    plsc.addupdate_scatter(
                        acc, [seg_row, jnp.full((16,), _HW, jnp.int32)],
                        ones, mask=mask0)
                    return carry
                lax.fori_loop(r_lo, r_hi, row, 0)

            def chunk_work(dbuf, bbuf, c):
                s2 = jnp.minimum(c * _CH, n_rows - _CH)
                lo = c * _CH - s2
                v = bbuf[pl.ds(0, 16)]
                bmin = v
                bmax = v
                for t in range(1, _CH // 16):
                    v = bbuf[pl.ds(t * 16, 16)]
                    bmin = jnp.minimum(bmin, v)
                    bmax = jnp.maximum(bmax, v)
                bmin_s = jnp.min(bmin)
                bmax_s = jnp.max(bmax)
                uniform = jnp.logical_and(lo == 0, bmin_s == bmax_s)

                def fast():
                    def body(i, carry):
                        av = list(carry)
                        r0 = i * _RU
                        for u in range(_RU):
                            for j in range(4):
                                av[j] = av[j] + dbuf[r0 + u,
                                                     pl.ds(j * 16, 16)]
                        return tuple(av)
                    av = lax.fori_loop(0, _CH // _RU, body,
                                       tuple([zeros] * 4))
                    flush_regs(av, jnp.full((16,), bmin_s + seg_base,
                                            jnp.int32),
                               jnp.full((16,), float(_CH), jnp.float32))

                def slow():
                    def sub(g, carry):
                        r0 = g * _RU
                        bvec = bbuf[pl.ds(r0, 16)]
                        big = jnp.full((16,), 2 ** 30, jnp.int32)
                        small = jnp.full((16,), -2 ** 30, jnp.int32)
                        sel = off < _RU
                        mn = jnp.min(jnp.where(sel, bvec, big))
                        mx = jnp.max(jnp.where(sel, bvec, small))
                        ok = jnp.logical_and(mn == mx, r0 >= lo)

                        def gfast():
                            av = [zeros] * 4
                            for u in range(_RU):
                                for j in range(4):
                                    av[j] = av[j] + dbuf[r0 + u,
                                                         pl.ds(j * 16, 16)]
                            flush_regs(av, jnp.full((16,), mn + seg_base,
                                                    jnp.int32),
                                       jnp.full((16,), float(_RU),
                                                jnp.float32))

                        def grows():
                            row_scatter(dbuf, bbuf, jnp.maximum(lo, r0),
                                        r0 + _RU)

                        lax.cond(ok, gfast, grows)
                        return carry
                    lax.fori_loop(lo // _RU, _CH // _RU, sub, 0)

                lax.cond(uniform, fast, slow)

            def body(k, carry):
                c = sid + k * _NT

                def run(cur, nxt):
                    pl.when(k + 1 < t_steps)(lambda: start_into(nxt, c + _NT))
                    wait_into(cur)
                    dbuf, bbuf, _ = cur
                    bbuf[pl.ds(_CH, 16)] = bbuf[pl.ds(_CH - 16, 16)]
                    pl.when(c < nchunks)(lambda: chunk_work(dbuf, bbuf, c))

                lax.cond(k % 2 == 0,
                         lambda: run(bufsA, bufsB),
                         lambda: run(bufsB, bufsA))
                return carry

            start_into(bufsA, sid)
            lax.fori_loop(0, t_steps, body, 0)

        process(cpg_h, cb_h, n_c, 0)
        process(mirna_h, mb_h, n_m, _B)

        pltpu.sync_copy(acc.at[pl.ds(0, _B)], szc.at[iref], add=True)
        pltpu.sync_copy(acc.at[pl.ds(_B, _B)], szm.at[iref], add=True)
        plsc.subcore_barrier()

        def emit(shared, out_h):
            pltpu.sync_copy(shared.at[sid], rowbuf)
            cntv = plsc.load_gather(rowbuf,
                                    [jnp.full((16,), _HW, jnp.int32)])
            den = jnp.maximum(cntv, 1.0)
            for j in range(4):
                sbuf[pl.ds(j * 16, 16)] = rowbuf[pl.ds(j * 16, 16)] / den
            pltpu.sync_copy(sbuf, out_h.at[sid, pl.ds(col0, _HW)])

        emit(szc, dna_h)
        emit(szm, mir_h)

    return sc_run(cpg2, cb, mirna2, mb)


def _gene_body(b_ref, x_ref, wmt_ref, wct_ref, bm_ref, bc_ref,
               mrna_ref, cnv_ref, gsum, gcnt):
    i = pl.program_id(0)
    k_steps = pl.num_programs(0)

    @pl.when(i == 0)
    def _init():
        gsum[...] = jnp.zeros_like(gsum)
        gcnt[...] = jnp.zeros_like(gcnt)

    seg_ids = lax.broadcasted_iota(jnp.int32, (_B, _RG), 0)
    oh = (seg_ids == b_ref[0]).astype(jnp.float32)
    gsum[...] += jnp.dot(oh, x_ref[...], preferred_element_type=jnp.float32)
    gcnt[...] += jnp.sum(oh, axis=1, keepdims=True)

    @pl.when(i == k_steps - 1)
    def _fin():
        cnt = gcnt[:, 0:1]
        mean = gsum[...] / jnp.maximum(cnt, 1.0)
        mask = (cnt > 0.0).astype(jnp.float32)
        dn = (((1,), (1,)), ((), ()))
        mrna_ref[...] = (lax.dot_general(mean, wmt_ref[...], dn,
                                         preferred_element_type=jnp.float32)
                         + bm_ref[...] * mask)
        cnv_ref[...] = (lax.dot_general(mean, wct_ref[...], dn,
                                        preferred_element_type=jnp.float32)
                        + bc_ref[...] * mask)


def _gene_project(gene, gb3, wmt, wct, bm2, bc2):
    k_steps = gene.shape[0] // _RG
    return pl.pallas_call(
        _gene_body,
        grid=(k_steps,),
        in_specs=[
            pl.BlockSpec((1, 1, _RG), lambda i: (i, 0, 0)),
            pl.BlockSpec((_RG, _H), lambda i: (i, 0)),
            pl.BlockSpec((_H, _H), lambda i: (0, 0)),
            pl.BlockSpec((_H, _H), lambda i: (0, 0)),
            pl.BlockSpec((1, _H), lambda i: (0, 0)),
            pl.BlockSpec((1, _H), lambda i: (0, 0)),
        ],
        out_specs=[
            pl.BlockSpec((_B, _H), lambda i: (0, 0)),
            pl.BlockSpec((_B, _H), lambda i: (0, 0)),
        ],
        out_shape=[jax.ShapeDtypeStruct((_B, _H), jnp.float32)] * 2,
        scratch_shapes=[
            pltpu.VMEM((_B, _H), jnp.float32),
            pltpu.VMEM((_B, _H), jnp.float32),
        ],
    )(gb3, gene, wmt, wct, bm2, bc2)


def kernel(gene, cpg, mirna, gene_batch, cpg_batch, mirna_batch, Wm, bm, Wc, bc):
    gene = gene.astype(jnp.float32)
    cpg = cpg.astype(jnp.float32)
    mirna = mirna.astype(jnp.float32)
    gb = gene_batch.astype(jnp.int32)
    cb = cpg_batch.astype(jnp.int32)
    mb = mirna_batch.astype(jnp.int32)

    dna, mir = _sc_pool_means(cpg, cb, mirna, mb)
    mrna, cnv = _gene_project(
        gene, gb.reshape(-1, 1, _RG),
        Wm.astype(jnp.float32), Wc.astype(jnp.float32),
        bm.astype(jnp.float32).reshape(1, _H),
        bc.astype(jnp.float32).reshape(1, _H),
    )
    return (mrna, cnv, dna, mir)
